# baseline (speedup 1.0000x reference)
import functools

import jax
import jax.numpy as jnp
from jax.experimental import pallas as pl
from jax.experimental.pallas import tpu as pltpu

_EPS = 1e-12
_ATOL = 1e-8
_RTOL = 1e-5
_CHUNK = 512


def _row_ss(x):
    d = x.shape[1]
    acc = None
    for c in range(0, d, _CHUNK):
        blk = x[:, c:c + _CHUNK]
        sq = blk * blk
        acc = sq if acc is None else acc + sq
    return jnp.sum(acc, axis=1, keepdims=True)


def _prologue_kernel(q_ref, t_ref, qn_ref, te_ref):
    q = q_ref[...]
    t = t_ref[...]
    qn = q / jnp.maximum(jnp.sqrt(_row_ss(q)), _EPS)
    tn = t / jnp.maximum(jnp.sqrt(_row_ss(t)), _EPS)
    qn_ref[...] = qn
    m = jax.lax.dot_general(qn, tn, (((1,), (1,)), ((), ())))
    nq = m.shape[0]
    eye = (jax.lax.broadcasted_iota(jnp.int32, (nq, nq), 0)
           == jax.lax.broadcasted_iota(jnp.int32, (nq, nq), 1))
    thr = jnp.sum(jnp.where(eye, m, 0.0), axis=0, keepdims=True)
    te_ref[...] = thr - (_ATOL + _RTOL * jnp.abs(thr))


def _count_kernel(qn_ref, te_ref, d_ref, out_ref):
    k = pl.program_id(0)
    d = d_ref[...]
    s = jax.lax.dot_general(d, qn_ref[...], (((1,), (1,)), ((), ())))
    norm = jnp.maximum(jnp.sqrt(_row_ss(d)), _EPS)
    ge = s >= te_ref[...] * norm
    cnt = jnp.sum(ge.astype(jnp.int32), axis=0, keepdims=True)

    @pl.when(k == 0)
    def _first():
        out_ref[...] = cnt - 1

    @pl.when(k != 0)
    def _rest():
        out_ref[...] = out_ref[...] + cnt


@functools.partial(jax.jit, static_argnames=("block",))
def _rank(queries, truths, data, block=512):
    n, d = data.shape
    nq = queries.shape[0]
    qn, te = pl.pallas_call(
        _prologue_kernel,
        out_shape=(
            jax.ShapeDtypeStruct((nq, d), jnp.float32),
            jax.ShapeDtypeStruct((1, nq), jnp.float32),
        ),
    )(queries, truths)
    return pl.pallas_call(
        _count_kernel,
        grid=(n // block,),
        in_specs=[
            pl.BlockSpec((nq, d), lambda k: (0, 0)),
            pl.BlockSpec((1, nq), lambda k: (0, 0)),
            pl.BlockSpec((block, d), lambda k: (k, 0)),
        ],
        out_specs=pl.BlockSpec((1, nq), lambda k: (0, 0)),
        out_shape=jax.ShapeDtypeStruct((1, nq), jnp.int32),
        compiler_params=pltpu.CompilerParams(
            dimension_semantics=("arbitrary",),
        ),
    )(qn, te, data)


def kernel(queries, truths, data):
    return _rank(queries, truths, data)


# speedup vs baseline: 1.0627x; 1.0627x over previous
import functools

import jax
import jax.numpy as jnp
from jax.experimental import pallas as pl
from jax.experimental.pallas import tpu as pltpu

_EPS = 1e-12
_ATOL = 1e-8
_RTOL = 1e-5
_CHUNK = 512


def _row_ss(x):
    d = x.shape[1]
    acc = None
    for c in range(0, d, _CHUNK):
        blk = x[:, c:c + _CHUNK]
        sq = blk * blk
        acc = sq if acc is None else acc + sq
    return jnp.sum(acc, axis=1, keepdims=True)


def _normalize_rows(x):
    return x / jnp.maximum(jnp.sqrt(_row_ss(x)), _EPS)


def _rank_kernel(q_ref, t_ref, da_ref, db_ref, out_ref, qn_ref, te_ref):
    k = pl.program_id(0)

    @pl.when(k == 0)
    def _init():
        qn = _normalize_rows(q_ref[...])
        tn = _normalize_rows(t_ref[...])
        qn_ref[...] = qn
        m = jax.lax.dot_general(qn, tn, (((1,), (1,)), ((), ())))
        nq = m.shape[0]
        eye = (jax.lax.broadcasted_iota(jnp.int32, (nq, nq), 0)
               == jax.lax.broadcasted_iota(jnp.int32, (nq, nq), 1))
        thr = jnp.sum(jnp.where(eye, m, 0.0), axis=0, keepdims=True)
        te_ref[...] = thr - (_ATOL + _RTOL * jnp.abs(thr))

    da = da_ref[...]
    db = db_ref[...]
    dh = da.shape[1]
    qn = qn_ref[...]
    s = (jax.lax.dot_general(da, qn[:, :dh], (((1,), (1,)), ((), ())))
         + jax.lax.dot_general(db, qn[:, dh:], (((1,), (1,)), ((), ()))))
    norm = jnp.maximum(jnp.sqrt(_row_ss(da) + _row_ss(db)), _EPS)
    ge = s >= te_ref[...] * norm
    cnt = jnp.sum(ge.astype(jnp.int32), axis=0, keepdims=True)

    @pl.when(k == 0)
    def _first():
        out_ref[...] = cnt - 1

    @pl.when(k != 0)
    def _rest():
        out_ref[...] = out_ref[...] + cnt


@functools.partial(jax.jit, static_argnames=("block",))
def _rank(queries, truths, data, block=512):
    n, d = data.shape
    nq = queries.shape[0]
    dh = d // 2
    grid = (n // block,)
    return pl.pallas_call(
        _rank_kernel,
        grid=grid,
        in_specs=[
            pl.BlockSpec((nq, d), lambda k: (0, 0)),
            pl.BlockSpec((nq, d), lambda k: (0, 0)),
            pl.BlockSpec((block, dh), lambda k: (k, 0)),
            pl.BlockSpec((block, dh), lambda k: (k, 1)),
        ],
        out_specs=pl.BlockSpec((1, nq), lambda k: (0, 0)),
        out_shape=jax.ShapeDtypeStruct((1, nq), jnp.int32),
        scratch_shapes=[
            pltpu.VMEM((nq, d), jnp.float32),
            pltpu.VMEM((1, nq), jnp.float32),
        ],
        compiler_params=pltpu.CompilerParams(
            dimension_semantics=("arbitrary",),
        ),
    )(queries, truths, data, data)


def kernel(queries, truths, data):
    return _rank(queries, truths, data)
